# Initial kernel scaffold; baseline (speedup 1.0000x reference)
import jax
import jax.numpy as jnp
from jax.experimental import pallas as pl


def kernel(x, edge_index, W_sage_l, b_sage_l, W_sage_r, W_gcn, b_gcn):
    raise NotImplementedError("write your pallas kernel here")


# speedup vs baseline: 21.9324x; 21.9324x over previous
import functools

import jax
import jax.numpy as jnp
from jax import lax
from jax.experimental import pallas as pl
from jax.experimental.pallas import tpu as pltpu
from jax.experimental.pallas import tpu_sc as plsc

N = 10000
E = 320000
DIN = 128
DH = 128
DOUT = 64

NC = 2
NS = 16
EB = 100
EPT = E // (NC * NS)
NB = EPT // EB
RPT = 624
REM = N - NS * RPT
DEGW = 16

_mesh = plsc.VectorSubcoreMesh(core_axis_name="c", subcore_axis_name="s")


def _fill_rows(ref, nrows, ncols, value):
    v = jnp.full((16,), value, jnp.float32)
    cpr = ncols // 16

    def body(i, carry):
        r = i // cpr
        col = (i % cpr) * 16
        ref[r, pl.ds(col, 16)] = v
        return carry

    lax.fori_loop(0, nrows * cpr, body, 0)


def _make_sc_agg(d, with_deg):
    out_type = [jax.ShapeDtypeStruct((NC * N, d), jnp.float32)]
    scratch = [
        pltpu.VMEM_SHARED((N, d), jnp.float32),
        pltpu.VMEM((NB, EB), jnp.int32),
        pltpu.VMEM((NB, EB), jnp.int32),
        pltpu.VMEM((EB, d), jnp.float32),
        pltpu.SemaphoreType.DMA,
    ]
    if with_deg:
        out_type.append(jax.ShapeDtypeStruct((NC * N, DEGW), jnp.float32))
        scratch += [
            pltpu.VMEM_SHARED((N, DEGW), jnp.float32),
            pltpu.VMEM((EB, DEGW), jnp.float32),
            pltpu.VMEM((EB, DEGW), jnp.float32),
        ]

    @functools.partial(
        pl.kernel, mesh=_mesh, out_type=out_type, scratch_types=scratch,
        compiler_params=pltpu.CompilerParams(use_tc_tiling_on_sc=False))
    def sc_agg(table_hbm, src_hbm, dst_hbm, *rest):
        if with_deg:
            (msg_hbm, deg_hbm, acc, src_v, dst_v, rows_v, sem,
             dacc, ones_v, dz_v) = rest
        else:
            msg_hbm, acc, src_v, dst_v, rows_v, sem = rest
        c = lax.axis_index("c")
        s = lax.axis_index("s")

        _fill_rows(rows_v, EB, d, 0.0)
        if with_deg:
            _fill_rows(ones_v, EB, DEGW, 1.0)
            _fill_rows(dz_v, EB, DEGW, 0.0)

        row0 = s * RPT
        zc = 78
        for k in range(RPT // zc):
            pltpu.sync_copy(rows_v.at[pl.ds(0, zc)],
                            acc.at[pl.ds(row0 + k * zc, zc)])
            if with_deg:
                pltpu.sync_copy(dz_v.at[pl.ds(0, zc)],
                                dacc.at[pl.ds(row0 + k * zc, zc)])

        @pl.when(s == NS - 1)
        def _zero_rem():
            pltpu.sync_copy(rows_v.at[pl.ds(0, REM)],
                            acc.at[pl.ds(NS * RPT, REM)])
            if with_deg:
                pltpu.sync_copy(dz_v.at[pl.ds(0, REM)],
                                dacc.at[pl.ds(NS * RPT, REM)])

        plsc.subcore_barrier()

        tid = c * NS + s
        pltpu.sync_copy(src_hbm.at[tid], src_v)
        pltpu.sync_copy(dst_hbm.at[tid], dst_v)

        def body(i, carry):
            pltpu.async_copy(table_hbm.at[src_v.at[i]], rows_v, sem).wait()
            pltpu.sync_copy(rows_v, acc.at[dst_v.at[i]], add=True)
            if with_deg:
                pltpu.sync_copy(ones_v, dacc.at[dst_v.at[i]], add=True)
            return carry

        lax.fori_loop(0, NB, body, 0)
        plsc.subcore_barrier()

        out0 = c * N + row0
        pltpu.sync_copy(acc.at[pl.ds(row0, RPT)], msg_hbm.at[pl.ds(out0, RPT)])
        if with_deg:
            pltpu.sync_copy(dacc.at[pl.ds(row0, RPT)],
                            deg_hbm.at[pl.ds(out0, RPT)])

        @pl.when(s == NS - 1)
        def _drain_rem():
            pltpu.sync_copy(acc.at[pl.ds(NS * RPT, REM)],
                            msg_hbm.at[pl.ds(c * N + NS * RPT, REM)])
            if with_deg:
                pltpu.sync_copy(dacc.at[pl.ds(NS * RPT, REM)],
                                deg_hbm.at[pl.ds(c * N + NS * RPT, REM)])

    return sc_agg


_sc_agg128 = _make_sc_agg(DIN, with_deg=True)
_sc_agg64 = _make_sc_agg(DOUT, with_deg=False)

BN = 2000


def _tc_dense_body(msg_ref, deg_ref, x_ref, wl_ref, bl_ref, wr_ref, wg_ref,
                   y_ref, dinv_ref):
    m = msg_ref[0] + msg_ref[1]
    deg = (deg_ref[0] + deg_ref[1])[:, 0:1]
    mean = m / jnp.maximum(deg, 1.0)
    dn = (((1,), (1,)), ((), ()))
    h = lax.dot_general(mean, wl_ref[...], dn,
                        preferred_element_type=jnp.float32)
    h = h + bl_ref[...]
    h = h + lax.dot_general(x_ref[...], wr_ref[...], dn,
                            preferred_element_type=jnp.float32)
    h = jnp.maximum(h, 0.0)
    xt = lax.dot_general(h, wg_ref[...], dn,
                         preferred_element_type=jnp.float32)
    dinv = lax.rsqrt(deg + 1.0)
    y_ref[...] = dinv * xt
    dinv_ref[...] = jnp.broadcast_to(dinv, (BN, DOUT))


_tc_dense = pl.pallas_call(
    _tc_dense_body,
    grid=(N // BN,),
    in_specs=[
        pl.BlockSpec((NC, BN, DIN), lambda i: (0, i, 0)),
        pl.BlockSpec((NC, BN, DEGW), lambda i: (0, i, 0)),
        pl.BlockSpec((BN, DIN), lambda i: (i, 0)),
        pl.BlockSpec((DH, DIN), lambda i: (0, 0)),
        pl.BlockSpec((1, DH), lambda i: (0, 0)),
        pl.BlockSpec((DH, DIN), lambda i: (0, 0)),
        pl.BlockSpec((DOUT, DH), lambda i: (0, 0)),
    ],
    out_specs=[
        pl.BlockSpec((BN, DOUT), lambda i: (i, 0)),
        pl.BlockSpec((BN, DOUT), lambda i: (i, 0)),
    ],
    out_shape=[
        jax.ShapeDtypeStruct((N, DOUT), jnp.float32),
        jax.ShapeDtypeStruct((N, DOUT), jnp.float32),
    ],
)


def _tc_final_body(agg_ref, y_ref, dinv_ref, bg_ref, out_ref, soft_ref):
    a = agg_ref[0] + agg_ref[1] + y_ref[...]
    out = dinv_ref[...] * a + bg_ref[...]
    m = jnp.max(out, axis=1, keepdims=True)
    e = jnp.exp(out - m)
    out_ref[...] = out
    soft_ref[...] = e / jnp.sum(e, axis=1, keepdims=True)


_tc_final = pl.pallas_call(
    _tc_final_body,
    grid=(N // BN,),
    in_specs=[
        pl.BlockSpec((NC, BN, DOUT), lambda i: (0, i, 0)),
        pl.BlockSpec((BN, DOUT), lambda i: (i, 0)),
        pl.BlockSpec((BN, DOUT), lambda i: (i, 0)),
        pl.BlockSpec((1, DOUT), lambda i: (0, 0)),
    ],
    out_specs=[
        pl.BlockSpec((BN, DOUT), lambda i: (i, 0)),
        pl.BlockSpec((BN, DOUT), lambda i: (i, 0)),
    ],
    out_shape=[
        jax.ShapeDtypeStruct((N, DOUT), jnp.float32),
        jax.ShapeDtypeStruct((N, DOUT), jnp.float32),
    ],
)


def kernel(x, edge_index, W_sage_l, b_sage_l, W_sage_r, W_gcn, b_gcn):
    src2d = edge_index[0].reshape(NC * NS, NB, EB)
    dst2d = edge_index[1].reshape(NC * NS, NB, EB)
    msg2, deg2 = _sc_agg128(x, src2d, dst2d)
    y, dinv = _tc_dense(msg2.reshape(NC, N, DIN), deg2.reshape(NC, N, DEGW),
                        x, W_sage_l, b_sage_l.reshape(1, DH), W_sage_r, W_gcn)
    agg2 = _sc_agg64(y, src2d, dst2d)[0]
    out, soft = _tc_final(agg2.reshape(NC, N, DOUT), y, dinv,
                          b_gcn.reshape(1, DOUT))
    return out, soft


# speedup vs baseline: 26.7579x; 1.2200x over previous
import jax
import jax.numpy as jnp
from jax import lax
from jax.experimental import pallas as pl
from jax.experimental.pallas import tpu as pltpu
from jax.experimental.pallas import tpu_sc as plsc

N = 10000
E = 320000
DIN = 128
DH = 128
DOUT = 64

NC = 2
NS = 16
EB = 100
EPT = E // (NC * NS)
NB = EPT // EB
CH = 10
NCH = NB // CH
RPT = 624
REM = N - NS * RPT
DEGW = 8

_mesh = plsc.VectorSubcoreMesh(core_axis_name="c", subcore_axis_name="s")


def _make_sc_agg(d, with_deg):
    out_type = [jax.ShapeDtypeStruct((NC * N, d), jnp.float32)]
    scratch = [
        pltpu.VMEM_SHARED((N, d), jnp.float32),
        pltpu.VMEM((2, CH, EB), jnp.int32),
        pltpu.VMEM((2, CH, EB), jnp.int32),
        pltpu.VMEM((2, EB, d), jnp.float32),
        pltpu.SemaphoreType.DMA,
        pltpu.SemaphoreType.DMA,
        pltpu.SemaphoreType.DMA,
        pltpu.SemaphoreType.DMA,
        pltpu.SemaphoreType.DMA,
        pltpu.SemaphoreType.DMA,
    ]
    if with_deg:
        out_type.append(jax.ShapeDtypeStruct((NC * N, DEGW), jnp.float32))
        scratch += [
            pltpu.VMEM_SHARED((N, DEGW), jnp.float32),
            pltpu.VMEM((EB, DEGW), jnp.float32),
            pltpu.SemaphoreType.DMA,
        ]

    def sc_agg(table_hbm, src_hbm, dst_hbm, zrow_hbm, zdeg_hbm, ones_hbm,
               *rest):
        if with_deg:
            (msg_hbm, deg_hbm, acc, src_v, dst_v, rows_v,
             gs0, gs1, ss0, ss1, cs_s, cs_d, dacc, ones_v, osem) = rest
        else:
            (msg_hbm, acc, src_v, dst_v, rows_v,
             gs0, gs1, ss0, ss1, cs_s, cs_d) = rest
        gsem = (gs0, gs1)
        ssem = (ss0, ss1)
        c = lax.axis_index("c")
        s = lax.axis_index("s")
        tid = c * NS + s

        row0 = s * RPT
        pltpu.sync_copy(zrow_hbm.at[pl.ds(0, RPT)], acc.at[pl.ds(row0, RPT)])
        if with_deg:
            pltpu.sync_copy(zdeg_hbm.at[pl.ds(0, RPT)],
                            dacc.at[pl.ds(row0, RPT)])
            pltpu.sync_copy(ones_hbm, ones_v)

        @pl.when(s == NS - 1)
        def _zero_rem():
            pltpu.sync_copy(zrow_hbm.at[pl.ds(0, REM)],
                            acc.at[pl.ds(NS * RPT, REM)])
            if with_deg:
                pltpu.sync_copy(zdeg_hbm.at[pl.ds(0, REM)],
                                dacc.at[pl.ds(NS * RPT, REM)])

        plsc.subcore_barrier()

        pend_scatter = [None, None]
        pend_gather = [None, None]
        pend_chunk = None
        pend_ones = None
        for g in range(NB):
            k, j = divmod(g, CH)
            kb = k % 2
            b = g % 2
            if j == 0:
                if k == 0:
                    pltpu.sync_copy(src_hbm.at[tid, pl.ds(0, CH)],
                                    src_v.at[0])
                    pltpu.sync_copy(dst_hbm.at[tid, pl.ds(0, CH)],
                                    dst_v.at[0])
                else:
                    for dsc in pend_chunk:
                        dsc.wait()
                    pend_chunk = None
                if k + 1 < NCH:
                    pend_chunk = (
                        pltpu.async_copy(
                            src_hbm.at[tid, pl.ds((k + 1) * CH, CH)],
                            src_v.at[1 - kb], cs_s),
                        pltpu.async_copy(
                            dst_hbm.at[tid, pl.ds((k + 1) * CH, CH)],
                            dst_v.at[1 - kb], cs_d),
                    )
            if pend_gather[b] is None:
                if pend_scatter[b] is not None:
                    pend_scatter[b].wait()
                    pend_scatter[b] = None
                pend_gather[b] = pltpu.async_copy(
                    table_hbm.at[src_v.at[kb].at[j]], rows_v.at[b], gsem[b])
            pend_gather[b].wait()
            pend_gather[b] = None
            if g + 1 < NB and (g + 1) // CH == k:
                if pend_scatter[1 - b] is not None:
                    pend_scatter[1 - b].wait()
                    pend_scatter[1 - b] = None
                pend_gather[1 - b] = pltpu.async_copy(
                    table_hbm.at[src_v.at[kb].at[j + 1]], rows_v.at[1 - b],
                    gsem[1 - b])
            pend_scatter[b] = pltpu.async_copy(
                rows_v.at[b], acc.at[dst_v.at[kb].at[j]], ssem[b], add=True)
            if with_deg:
                if pend_ones is not None:
                    pend_ones.wait()
                pend_ones = pltpu.async_copy(
                    ones_v, dacc.at[dst_v.at[kb].at[j]], osem, add=True)
        for p in pend_scatter:
            if p is not None:
                p.wait()
        if pend_ones is not None:
            pend_ones.wait()
        plsc.subcore_barrier()

        out0 = c * N + row0
        pltpu.sync_copy(acc.at[pl.ds(row0, RPT)], msg_hbm.at[pl.ds(out0, RPT)])
        if with_deg:
            pltpu.sync_copy(dacc.at[pl.ds(row0, RPT)],
                            deg_hbm.at[pl.ds(out0, RPT)])

        @pl.when(s == NS - 1)
        def _drain_rem():
            pltpu.sync_copy(acc.at[pl.ds(NS * RPT, REM)],
                            msg_hbm.at[pl.ds(c * N + NS * RPT, REM)])
            if with_deg:
                pltpu.sync_copy(dacc.at[pl.ds(NS * RPT, REM)],
                                deg_hbm.at[pl.ds(c * N + NS * RPT, REM)])

    return pl.kernel(
        sc_agg, mesh=_mesh, out_type=out_type, scratch_types=scratch,
        compiler_params=pltpu.CompilerParams(use_tc_tiling_on_sc=False))


_sc_agg128 = _make_sc_agg(DIN, with_deg=True)
_sc_agg64 = _make_sc_agg(DOUT, with_deg=False)

BN = 2000


def _tc_dense_body(msg_ref, deg_ref, x_ref, wl_ref, bl_ref, wr_ref, wg_ref,
                   y_ref, dinv_ref):
    m = msg_ref[0] + msg_ref[1]
    deg = (deg_ref[0] + deg_ref[1])[:, 0:1]
    mean = m / jnp.maximum(deg, 1.0)
    dn = (((1,), (1,)), ((), ()))
    h = lax.dot_general(mean, wl_ref[...], dn,
                        preferred_element_type=jnp.float32)
    h = h + bl_ref[...]
    h = h + lax.dot_general(x_ref[...], wr_ref[...], dn,
                            preferred_element_type=jnp.float32)
    h = jnp.maximum(h, 0.0)
    xt = lax.dot_general(h, wg_ref[...], dn,
                         preferred_element_type=jnp.float32)
    dinv = lax.rsqrt(deg + 1.0)
    y_ref[...] = dinv * xt
    dinv_ref[...] = jnp.broadcast_to(dinv, (BN, DOUT))


_tc_dense = pl.pallas_call(
    _tc_dense_body,
    grid=(N // BN,),
    in_specs=[
        pl.BlockSpec((NC, BN, DIN), lambda i: (0, i, 0)),
        pl.BlockSpec((NC, BN, DEGW), lambda i: (0, i, 0)),
        pl.BlockSpec((BN, DIN), lambda i: (i, 0)),
        pl.BlockSpec((DH, DIN), lambda i: (0, 0)),
        pl.BlockSpec((1, DH), lambda i: (0, 0)),
        pl.BlockSpec((DH, DIN), lambda i: (0, 0)),
        pl.BlockSpec((DOUT, DH), lambda i: (0, 0)),
    ],
    out_specs=[
        pl.BlockSpec((BN, DOUT), lambda i: (i, 0)),
        pl.BlockSpec((BN, DOUT), lambda i: (i, 0)),
    ],
    out_shape=[
        jax.ShapeDtypeStruct((N, DOUT), jnp.float32),
        jax.ShapeDtypeStruct((N, DOUT), jnp.float32),
    ],
)


def _tc_final_body(agg_ref, y_ref, dinv_ref, bg_ref, out_ref, soft_ref):
    a = agg_ref[0] + agg_ref[1] + y_ref[...]
    out = dinv_ref[...] * a + bg_ref[...]
    m = jnp.max(out, axis=1, keepdims=True)
    e = jnp.exp(out - m)
    out_ref[...] = out
    soft_ref[...] = e / jnp.sum(e, axis=1, keepdims=True)


_tc_final = pl.pallas_call(
    _tc_final_body,
    grid=(N // BN,),
    in_specs=[
        pl.BlockSpec((NC, BN, DOUT), lambda i: (0, i, 0)),
        pl.BlockSpec((BN, DOUT), lambda i: (i, 0)),
        pl.BlockSpec((BN, DOUT), lambda i: (i, 0)),
        pl.BlockSpec((1, DOUT), lambda i: (0, 0)),
    ],
    out_specs=[
        pl.BlockSpec((BN, DOUT), lambda i: (i, 0)),
        pl.BlockSpec((BN, DOUT), lambda i: (i, 0)),
    ],
    out_shape=[
        jax.ShapeDtypeStruct((N, DOUT), jnp.float32),
        jax.ShapeDtypeStruct((N, DOUT), jnp.float32),
    ],
)


def kernel(x, edge_index, W_sage_l, b_sage_l, W_sage_r, W_gcn, b_gcn):
    src3d = edge_index[0].reshape(NC * NS, NB, EB)
    dst3d = edge_index[1].reshape(NC * NS, NB, EB)
    zrow = jnp.zeros((RPT, DIN), jnp.float32)
    zdeg = jnp.zeros((RPT, DEGW), jnp.float32)
    ones = jnp.ones((EB, DEGW), jnp.float32)
    msg2, deg2 = _sc_agg128(x, src3d, dst3d, zrow, zdeg, ones)
    y, dinv = _tc_dense(msg2.reshape(NC, N, DIN), deg2.reshape(NC, N, DEGW),
                        x, W_sage_l, b_sage_l.reshape(1, DH), W_sage_r, W_gcn)
    zrow64 = jnp.zeros((RPT, DOUT), jnp.float32)
    agg2 = _sc_agg64(y, src3d, dst3d, zrow64, zdeg, ones)[0]
    out, soft = _tc_final(agg2.reshape(NC, N, DOUT), y, dinv,
                          b_gcn.reshape(1, DOUT))
    return out, soft


# speedup vs baseline: 35.8357x; 1.3393x over previous
import jax
import jax.numpy as jnp
from jax import lax
from jax.experimental import pallas as pl
from jax.experimental.pallas import tpu as pltpu
from jax.experimental.pallas import tpu_sc as plsc

N = 10000
E = 320000
DIN = 128
DH = 128
DOUT = 64

NC = 2
NS = 16
EPT = E // (NC * NS)
RPT = 624
REM = N - NS * RPT
DEGW = 8

_mesh = plsc.VectorSubcoreMesh(core_axis_name="c", subcore_axis_name="s")


def _make_sc_agg(d, with_deg, eb, nbuf, ch):
    nb = EPT // eb
    nch = nb // ch
    out_type = [jax.ShapeDtypeStruct((NC * N, d), jnp.float32)]
    scratch = [
        pltpu.VMEM_SHARED((N, d), jnp.float32),
        pltpu.VMEM((2, ch, eb), jnp.int32),
        pltpu.VMEM((2, ch, eb), jnp.int32),
        pltpu.VMEM((nbuf, eb, d), jnp.float32),
        pltpu.SemaphoreType.DMA,
        pltpu.SemaphoreType.DMA,
    ]
    scratch += [pltpu.SemaphoreType.DMA] * (2 * nbuf)
    if with_deg:
        out_type.append(jax.ShapeDtypeStruct((NC * N, DEGW), jnp.float32))
        scratch += [
            pltpu.VMEM_SHARED((N, DEGW), jnp.float32),
            pltpu.VMEM((eb, DEGW), jnp.float32),
            pltpu.SemaphoreType.DMA,
        ]

    def sc_agg(table_hbm, src_hbm, dst_hbm, zrow_hbm, zdeg_hbm, ones_hbm,
               *rest):
        if with_deg:
            (msg_hbm, deg_hbm, acc, src_v, dst_v, rows_v, cs_s, cs_d,
             *sems) = rest
            sems, (dacc, ones_v, osem) = sems[:2 * nbuf], sems[2 * nbuf:]
        else:
            (msg_hbm, acc, src_v, dst_v, rows_v, cs_s, cs_d, *sems) = rest
        gsem = sems[:nbuf]
        ssem = sems[nbuf:2 * nbuf]
        c = lax.axis_index("c")
        s = lax.axis_index("s")
        tid = c * NS + s

        row0 = s * RPT
        pltpu.sync_copy(zrow_hbm.at[pl.ds(0, RPT)], acc.at[pl.ds(row0, RPT)])
        if with_deg:
            pltpu.sync_copy(zdeg_hbm.at[pl.ds(0, RPT)],
                            dacc.at[pl.ds(row0, RPT)])
            pltpu.sync_copy(ones_hbm, ones_v)

        @pl.when(s == NS - 1)
        def _zero_rem():
            pltpu.sync_copy(zrow_hbm.at[pl.ds(0, REM)],
                            acc.at[pl.ds(NS * RPT, REM)])
            if with_deg:
                pltpu.sync_copy(zdeg_hbm.at[pl.ds(0, REM)],
                                dacc.at[pl.ds(NS * RPT, REM)])

        plsc.subcore_barrier()

        pend_scatter = [None] * nbuf
        pend_gather = [None] * nbuf
        pend_chunk = None
        pend_ones = None

        def fire_gather(gn, kb):
            bn = gn % nbuf
            if pend_scatter[bn] is not None:
                pend_scatter[bn].wait()
                pend_scatter[bn] = None
            pend_gather[bn] = pltpu.async_copy(
                table_hbm.at[src_v.at[kb].at[gn % ch]], rows_v.at[bn],
                gsem[bn])

        for g in range(nb):
            k, j = divmod(g, ch)
            kb = k % 2
            b = g % nbuf
            if j == 0:
                if k == 0:
                    pltpu.sync_copy(src_hbm.at[tid, pl.ds(0, ch)],
                                    src_v.at[0])
                    pltpu.sync_copy(dst_hbm.at[tid, pl.ds(0, ch)],
                                    dst_v.at[0])
                else:
                    for dsc in pend_chunk:
                        dsc.wait()
                    pend_chunk = None
                if k + 1 < nch:
                    pend_chunk = (
                        pltpu.async_copy(
                            src_hbm.at[tid, pl.ds((k + 1) * ch, ch)],
                            src_v.at[1 - kb], cs_s),
                        pltpu.async_copy(
                            dst_hbm.at[tid, pl.ds((k + 1) * ch, ch)],
                            dst_v.at[1 - kb], cs_d),
                    )
            if pend_gather[b] is None:
                fire_gather(g, kb)
            pend_gather[b].wait()
            pend_gather[b] = None
            for a in range(1, nbuf):
                gn = g + a
                if gn < nb and gn // ch == k and pend_gather[gn % nbuf] is None:
                    fire_gather(gn, kb)
            pend_scatter[b] = pltpu.async_copy(
                rows_v.at[b], acc.at[dst_v.at[kb].at[j]], ssem[b], add=True)
            if with_deg:
                if pend_ones is not None:
                    pend_ones.wait()
                pend_ones = pltpu.async_copy(
                    ones_v, dacc.at[dst_v.at[kb].at[j]], osem, add=True)
        for p in pend_scatter:
            if p is not None:
                p.wait()
        if pend_ones is not None:
            pend_ones.wait()
        plsc.subcore_barrier()

        out0 = c * N + row0
        pltpu.sync_copy(acc.at[pl.ds(row0, RPT)], msg_hbm.at[pl.ds(out0, RPT)])
        if with_deg:
            pltpu.sync_copy(dacc.at[pl.ds(row0, RPT)],
                            deg_hbm.at[pl.ds(out0, RPT)])

        @pl.when(s == NS - 1)
        def _drain_rem():
            pltpu.sync_copy(acc.at[pl.ds(NS * RPT, REM)],
                            msg_hbm.at[pl.ds(c * N + NS * RPT, REM)])
            if with_deg:
                pltpu.sync_copy(dacc.at[pl.ds(NS * RPT, REM)],
                                deg_hbm.at[pl.ds(c * N + NS * RPT, REM)])

    return pl.kernel(
        sc_agg, mesh=_mesh, out_type=out_type, scratch_types=scratch,
        compiler_params=pltpu.CompilerParams(use_tc_tiling_on_sc=False))


EB1, NBUF1, CH1 = 80, 3, 25
EB2, NBUF2, CH2 = 125, 4, 20
_sc_agg128 = _make_sc_agg(DIN, True, EB1, NBUF1, CH1)
_sc_agg64 = _make_sc_agg(DOUT, False, EB2, NBUF2, CH2)

BN = 2000


def _tc_dense_body(msg_ref, deg_ref, x_ref, wl_ref, bl_ref, wr_ref, wg_ref,
                   y_ref, dinv_ref):
    m = msg_ref[0] + msg_ref[1]
    deg = (deg_ref[0] + deg_ref[1])[:, 0:1]
    mean = m / jnp.maximum(deg, 1.0)
    dn = (((1,), (1,)), ((), ()))
    h = lax.dot_general(mean, wl_ref[...], dn,
                        preferred_element_type=jnp.float32)
    h = h + bl_ref[...]
    h = h + lax.dot_general(x_ref[...], wr_ref[...], dn,
                            preferred_element_type=jnp.float32)
    h = jnp.maximum(h, 0.0)
    xt = lax.dot_general(h, wg_ref[...], dn,
                         preferred_element_type=jnp.float32)
    dinv = lax.rsqrt(deg + 1.0)
    y_ref[...] = dinv * xt
    dinv_ref[...] = jnp.broadcast_to(dinv, (BN, DOUT))


_tc_dense = pl.pallas_call(
    _tc_dense_body,
    grid=(N // BN,),
    in_specs=[
        pl.BlockSpec((NC, BN, DIN), lambda i: (0, i, 0)),
        pl.BlockSpec((NC, BN, DEGW), lambda i: (0, i, 0)),
        pl.BlockSpec((BN, DIN), lambda i: (i, 0)),
        pl.BlockSpec((DH, DIN), lambda i: (0, 0)),
        pl.BlockSpec((1, DH), lambda i: (0, 0)),
        pl.BlockSpec((DH, DIN), lambda i: (0, 0)),
        pl.BlockSpec((DOUT, DH), lambda i: (0, 0)),
    ],
    out_specs=[
        pl.BlockSpec((BN, DOUT), lambda i: (i, 0)),
        pl.BlockSpec((BN, DOUT), lambda i: (i, 0)),
    ],
    out_shape=[
        jax.ShapeDtypeStruct((N, DOUT), jnp.float32),
        jax.ShapeDtypeStruct((N, DOUT), jnp.float32),
    ],
)


def _tc_final_body(agg_ref, y_ref, dinv_ref, bg_ref, out_ref, soft_ref):
    a = agg_ref[0] + agg_ref[1] + y_ref[...]
    out = dinv_ref[...] * a + bg_ref[...]
    m = jnp.max(out, axis=1, keepdims=True)
    e = jnp.exp(out - m)
    out_ref[...] = out
    soft_ref[...] = e / jnp.sum(e, axis=1, keepdims=True)


_tc_final = pl.pallas_call(
    _tc_final_body,
    grid=(N // BN,),
    in_specs=[
        pl.BlockSpec((NC, BN, DOUT), lambda i: (0, i, 0)),
        pl.BlockSpec((BN, DOUT), lambda i: (i, 0)),
        pl.BlockSpec((BN, DOUT), lambda i: (i, 0)),
        pl.BlockSpec((1, DOUT), lambda i: (0, 0)),
    ],
    out_specs=[
        pl.BlockSpec((BN, DOUT), lambda i: (i, 0)),
        pl.BlockSpec((BN, DOUT), lambda i: (i, 0)),
    ],
    out_shape=[
        jax.ShapeDtypeStruct((N, DOUT), jnp.float32),
        jax.ShapeDtypeStruct((N, DOUT), jnp.float32),
    ],
)


def kernel(x, edge_index, W_sage_l, b_sage_l, W_sage_r, W_gcn, b_gcn):
    src1 = edge_index[0].reshape(NC * NS, EPT // EB1, EB1)
    dst1 = edge_index[1].reshape(NC * NS, EPT // EB1, EB1)
    src2 = edge_index[0].reshape(NC * NS, EPT // EB2, EB2)
    dst2 = edge_index[1].reshape(NC * NS, EPT // EB2, EB2)
    zrow = jnp.zeros((RPT, DIN), jnp.float32)
    zdeg = jnp.zeros((RPT, DEGW), jnp.float32)
    ones = jnp.ones((EB1, DEGW), jnp.float32)
    msg2, deg2 = _sc_agg128(x, src1, dst1, zrow, zdeg, ones)
    y, dinv = _tc_dense(msg2.reshape(NC, N, DIN), deg2.reshape(NC, N, DEGW),
                        x, W_sage_l, b_sage_l.reshape(1, DH), W_sage_r, W_gcn)
    zrow64 = jnp.zeros((RPT, DOUT), jnp.float32)
    agg2 = _sc_agg64(y, src2, dst2, zrow64, zdeg, ones)[0]
    out, soft = _tc_final(agg2.reshape(NC, N, DOUT), y, dinv,
                          b_gcn.reshape(1, DOUT))
    return out, soft
